# baseline (speedup 1.0000x reference)
import functools

import jax
import jax.numpy as jnp
from jax import lax
from jax.experimental import pallas as pl
from jax.experimental.pallas import tpu as pltpu
from jax.experimental.pallas import tpu_sc as plsc

N = 10000
E = 320000
D = 128
H = 4
DH = D // H
HP = 16

NC = 2
NS = 16
NW = NC * NS
L = 16

EPW = E // NW
SB = 25
SUB = 4
CB = SUB * SB
NCH = EPW // CB
NPAIR = NCH // 2
RPW = EPW // SB
NWB = 10
NPS = N // NWB
INV = 1.0 / float(DH) ** 0.5

_mesh = plsc.VectorSubcoreMesh(core_axis_name="c", subcore_axis_name="s")
_sc_params = pltpu.CompilerParams(needs_layout_passes=False,
                                  use_tc_tiling_on_sc=False)


SUB_B = 1
CB_B = SUB_B * SB
NCH_B = EPW // CB_B
NPAIR_B = NCH_B // 2


def _drain(sem, dummy_hbm_src, dst):
    pltpu.make_async_copy(dummy_hbm_src, dst, sem).wait()


@functools.partial(
    pl.kernel,
    out_type=(
        jax.ShapeDtypeStruct((E, HP), jnp.float32),
        jax.ShapeDtypeStruct((NC * N, HP), jnp.float32),
    ),
    mesh=_mesh,
    compiler_params=_sc_params,
    scratch_types=[
        pltpu.VMEM((RPW, SB), jnp.int32),
        pltpu.VMEM((RPW, SB), jnp.int32),
        pltpu.VMEM((2, CB, D), jnp.float32),
        pltpu.VMEM((2, CB, D), jnp.float32),
        pltpu.VMEM((2 * CB, HP), jnp.float32),
        pltpu.VMEM_SHARED((N, HP), jnp.float32),
        pltpu.SemaphoreType.DMA,
        pltpu.SemaphoreType.DMA,
        pltpu.SemaphoreType.DMA,
        pltpu.SemaphoreType.DMA,
        pltpu.SemaphoreType.DMA,
    ],
)
def _edge_scores(q_hbm, k_hbm, dsts_hbm, srcs_hbm, zero4_hbm,
                 ex_hbm, den_hbm,
                 dsti, srci, qbuf, kbuf, exbuf, den_sp,
                 semg0, semg1, sems0, sems1, seme):
    cid = lax.axis_index("c")
    sid = lax.axis_index("s")
    wid = sid * NC + cid
    semg = (semg0, semg1)
    sems = (sems0, sems1)

    @pl.when(sid < NWB)
    def _init():
        pltpu.sync_copy(zero4_hbm, den_sp.at[pl.ds(sid * NPS, NPS)])

    lane = lax.iota(jnp.int32, L)
    pltpu.sync_copy(dsts_hbm.at[pl.ds(wid * RPW, RPW)], dsti)
    pltpu.sync_copy(srcs_hbm.at[pl.ds(wid * RPW, RPW)], srci)
    plsc.subcore_barrier()

    def fire_gathers(i, b):
        for j in range(SUB):
            row = i * SUB + j
            pltpu.async_copy(q_hbm.at[dsti.at[row]],
                             qbuf.at[b, pl.ds(j * SB, SB)], semg[b])
            pltpu.async_copy(k_hbm.at[srci.at[row]],
                             kbuf.at[b, pl.ds(j * SB, SB)], semg[b])

    def compute(i, b):
        @plsc.parallel_loop(0, CB, step=1, unroll=8)
        def edge(e):
            p = [qbuf[b, e, pl.ds(16 * t, L)] * kbuf[b, e, pl.ds(16 * t, L)]
                 for t in range(D // L)]
            sv = jnp.zeros((L,), jnp.float32)
            for h in range(H):
                sv = jnp.where(lane == h,
                               jnp.full((L,), jnp.sum(p[2 * h] + p[2 * h + 1])),
                               sv)
            exbuf[b * CB + e] = jnp.where(lane < H, jnp.exp(sv), 0.0)

    def fire_adds(i, b):
        for j in range(SUB):
            pltpu.async_copy(exbuf.at[pl.ds(b * CB + j * SB, SB)],
                             den_sp.at[dsti.at[i * SUB + j]], sems[b],
                             add=True)

    def drain_gathers(b):
        for j in range(SUB):
            _drain(semg[b], q_hbm.at[pl.ds(0, SB)],
                   qbuf.at[b, pl.ds(j * SB, SB)])
            _drain(semg[b], q_hbm.at[pl.ds(0, SB)],
                   kbuf.at[b, pl.ds(j * SB, SB)])

    def drain_adds(b):
        for j in range(SUB):
            _drain(sems[b], ex_hbm.at[pl.ds(0, SB)],
                   exbuf.at[pl.ds(b * CB + j * SB, SB)])

    fire_gathers(0, 0)

    def pair(jp, carry):
        i0 = jp * 2
        fire_gathers(i0 + 1, 1)

        @pl.when(jp >= 1)
        def _d0():
            drain_adds(0)
            _drain(seme, ex_hbm.at[pl.ds(0, 2 * CB)], exbuf)

        drain_gathers(0)
        compute(i0, 0)
        fire_adds(i0, 0)
        @pl.when(jp <= NPAIR - 2)
        def _g1():
            fire_gathers(i0 + 2, 0)

        @pl.when(jp >= 1)
        def _d1():
            drain_adds(1)

        drain_gathers(1)
        compute(i0 + 1, 1)
        fire_adds(i0 + 1, 1)
        pltpu.async_copy(exbuf, ex_hbm.at[pl.ds(wid * EPW + i0 * CB, 2 * CB)],
                         seme)
        return carry

    lax.fori_loop(0, NPAIR, pair, 0)
    drain_adds(0)
    drain_adds(1)
    _drain(seme, ex_hbm.at[pl.ds(0, 2 * CB)], exbuf)
    plsc.subcore_barrier()

    @pl.when(sid < NWB)
    def _writeback():
        pltpu.sync_copy(den_sp.at[pl.ds(sid * NPS, NPS)],
                        den_hbm.at[pl.ds(cid * N + sid * NPS, NPS)])


@functools.partial(
    pl.kernel,
    out_type=jax.ShapeDtypeStruct((NC * N, D), jnp.float32),
    mesh=_mesh,
    compiler_params=_sc_params,
    scratch_types=[
        pltpu.VMEM((RPW, SB), jnp.int32),
        pltpu.VMEM((RPW, SB), jnp.int32),
        pltpu.VMEM((2, CB_B, D), jnp.float32),
        pltpu.VMEM((2, CB_B, D), jnp.float32),
        pltpu.VMEM((2 * CB_B * HP,), jnp.float32),
        pltpu.VMEM_SHARED((N, D), jnp.float32),
        pltpu.SemaphoreType.DMA,
        pltpu.SemaphoreType.DMA,
        pltpu.SemaphoreType.DMA,
        pltpu.SemaphoreType.DMA,
    ],
)
def _aggregate(v_hbm, dsts_hbm, srcs_hbm, exflat_hbm, zero128_hbm,
               agg_hbm,
               dsti, srci, vbuf, msgbuf, exbuf, agg_sp,
               semg0, semg1, sems0, sems1):
    cid = lax.axis_index("c")
    sid = lax.axis_index("s")
    wid = sid * NC + cid
    semg = (semg0, semg1)
    sems = (sems0, sems1)

    @pl.when(sid < NWB)
    def _init():
        pltpu.sync_copy(zero128_hbm, agg_sp.at[pl.ds(sid * NPS, NPS)])

    pltpu.sync_copy(dsts_hbm.at[pl.ds(wid * RPW, RPW)], dsti)
    pltpu.sync_copy(srcs_hbm.at[pl.ds(wid * RPW, RPW)], srci)
    plsc.subcore_barrier()

    def fire_gathers(i, b):
        for j in range(SUB_B):
            pltpu.async_copy(v_hbm.at[srci.at[i * SUB_B + j]],
                             vbuf.at[b, pl.ds(j * SB, SB)], semg[b])
        pltpu.async_copy(
            exflat_hbm.at[pl.ds((wid * EPW + i * CB_B) * HP, CB_B * HP)],
            exbuf.at[pl.ds(b * CB_B * HP, CB_B * HP)], semg[b])

    def compute(i, b):
        @plsc.parallel_loop(0, CB_B, step=1, unroll=8)
        def edge(e):
            ev = exbuf[pl.ds((b * CB_B + e) * HP, L)]
            for h in range(H):
                a = jnp.full((L,), ev[h])
                for t2 in range(2):
                    c0 = h * DH + t2 * L
                    msgbuf[b, e, pl.ds(c0, L)] = vbuf[b, e, pl.ds(c0, L)] * a

    def fire_adds(i, b):
        for j in range(SUB_B):
            pltpu.async_copy(msgbuf.at[b, pl.ds(j * SB, SB)],
                             agg_sp.at[dsti.at[i * SUB_B + j]], sems[b],
                             add=True)

    def drain_gathers(b):
        for j in range(SUB_B):
            _drain(semg[b], v_hbm.at[pl.ds(0, SB)],
                   vbuf.at[b, pl.ds(j * SB, SB)])
        _drain(semg[b], exflat_hbm.at[pl.ds(0, CB_B * HP)],
               exbuf.at[pl.ds(b * CB_B * HP, CB_B * HP)])

    def drain_adds(b):
        for j in range(SUB_B):
            _drain(sems[b], v_hbm.at[pl.ds(0, SB)],
                   msgbuf.at[b, pl.ds(j * SB, SB)])

    fire_gathers(0, 0)

    def pair(jp, carry):
        i0 = jp * 2
        fire_gathers(i0 + 1, 1)

        @pl.when(jp >= 1)
        def _d0():
            drain_adds(0)

        drain_gathers(0)
        compute(i0, 0)
        fire_adds(i0, 0)

        @pl.when(jp <= NPAIR_B - 2)
        def _g1():
            fire_gathers(i0 + 2, 0)

        @pl.when(jp >= 1)
        def _d1():
            drain_adds(1)

        drain_gathers(1)
        compute(i0 + 1, 1)
        fire_adds(i0 + 1, 1)
        return carry

    lax.fori_loop(0, NPAIR_B, pair, 0)
    drain_adds(0)
    drain_adds(1)
    plsc.subcore_barrier()

    @pl.when(sid < NWB)
    def _writeback():
        pltpu.sync_copy(agg_sp.at[pl.ds(sid * NPS, NPS)],
                        agg_hbm.at[pl.ds(cid * N + sid * NPS, NPS)])


BR = 1000


def _qkv_body(x_ref, wq_ref, wk_ref, wv_ref, q_ref, k_ref, v_ref):
    xb = x_ref[...]
    q_ref[...] = jnp.dot(xb, wq_ref[...],
                         preferred_element_type=jnp.float32) * INV
    k_ref[...] = jnp.dot(xb, wk_ref[...], preferred_element_type=jnp.float32)
    v_ref[...] = jnp.dot(xb, wv_ref[...], preferred_element_type=jnp.float32)


_qkv_call = pl.pallas_call(
    _qkv_body,
    grid=(N // BR,),
    in_specs=[pl.BlockSpec((BR, D), lambda i: (i, 0))]
    + [pl.BlockSpec((D, D), lambda i: (0, 0))] * 3,
    out_specs=[pl.BlockSpec((BR, D), lambda i: (i, 0))] * 3,
    out_shape=[jax.ShapeDtypeStruct((N, D), jnp.float32)] * 3,
)


def _densum_body(d_ref, o_ref):
    o_ref[...] = d_ref[0] + d_ref[1]


_densum_call = pl.pallas_call(
    _densum_body,
    in_specs=[pl.BlockSpec((NC, N * HP // D, D), lambda: (0, 0, 0))],
    out_specs=pl.BlockSpec((N * HP // D, D), lambda: (0, 0)),
    out_shape=jax.ShapeDtypeStruct((N * HP // D, D), jnp.float32),
)


def _out_body(x_ref, a0_ref, a1_ref, d_ref, wo_ref, o_ref):
    row_h = lax.broadcasted_iota(jnp.int32, (HP, D), 0)
    col_h = lax.broadcasted_iota(jnp.int32, (HP, D), 1) // DH
    bmat = jnp.where(row_h == col_h, 1.0, 0.0).astype(jnp.float32)
    r = 1.0 / (d_ref[...] + 1e-9)
    dens = jnp.dot(r, bmat, preferred_element_type=jnp.float32)
    agg = (a0_ref[...] + a1_ref[...]) * dens
    o_ref[...] = x_ref[...] + jnp.dot(agg, wo_ref[...],
                                      preferred_element_type=jnp.float32)


_out_call = pl.pallas_call(
    _out_body,
    grid=(N // BR,),
    in_specs=[
        pl.BlockSpec((BR, D), lambda i: (i, 0)),
        pl.BlockSpec((BR, D), lambda i: (i, 0)),
        pl.BlockSpec((BR, D), lambda i: (i + N // BR, 0)),
        pl.BlockSpec((BR, HP), lambda i: (i, 0)),
        pl.BlockSpec((D, D), lambda i: (0, 0)),
    ],
    out_specs=pl.BlockSpec((BR, D), lambda i: (i, 0)),
    out_shape=jax.ShapeDtypeStruct((N, D), jnp.float32),
)


def kernel(x, Wq, Wk, Wv, Wo, edge_index):
    q, k, v = _qkv_call(x, Wq, Wk, Wv)
    srcs = edge_index[0].reshape(E // SB, SB)
    dsts = edge_index[1].reshape(E // SB, SB)
    zero4 = jnp.zeros((NPS, HP), jnp.float32)
    zero128 = jnp.zeros((NPS, D), jnp.float32)
    ex, den = _edge_scores(q, k, dsts, srcs, zero4)
    densum = _densum_call(den.reshape(NC, N * HP // D, D)).reshape(N, HP)
    agg = _aggregate(v, dsts, srcs, ex.reshape(-1), zero128)
    return _out_call(x, agg, agg, densum, Wo)


# speedup vs baseline: 1.0045x; 1.0045x over previous
import functools

import jax
import jax.numpy as jnp
from jax import lax
from jax.experimental import pallas as pl
from jax.experimental.pallas import tpu as pltpu
from jax.experimental.pallas import tpu_sc as plsc

N = 10000
E = 320000
D = 128
H = 4
DH = D // H
HP = 16

NC = 2
NS = 16
NW = NC * NS
L = 16

EPW = E // NW
SB = 25
SUB = 4
CB = SUB * SB
NCH = EPW // CB
NPAIR = NCH // 2
RPW = EPW // SB
NWB = 10
NPS = N // NWB
INV = 1.0 / float(DH) ** 0.5

_mesh = plsc.VectorSubcoreMesh(core_axis_name="c", subcore_axis_name="s")
_sc_params = pltpu.CompilerParams(needs_layout_passes=False,
                                  use_tc_tiling_on_sc=False)


SUB_B = 1
CB_B = SUB_B * SB
NCH_B = EPW // CB_B
NPAIR_B = NCH_B // 2


def _drain(sem, dummy_hbm_src, dst):
    pltpu.make_async_copy(dummy_hbm_src, dst, sem).wait()


@functools.partial(
    pl.kernel,
    out_type=(
        jax.ShapeDtypeStruct((E, HP), jnp.float32),
        jax.ShapeDtypeStruct((NC * N, HP), jnp.float32),
    ),
    mesh=_mesh,
    compiler_params=_sc_params,
    scratch_types=[
        pltpu.VMEM((RPW, SB), jnp.int32),
        pltpu.VMEM((RPW, SB), jnp.int32),
        pltpu.VMEM((2, CB, D), jnp.float32),
        pltpu.VMEM((2, CB, D), jnp.float32),
        pltpu.VMEM((2 * CB, HP), jnp.float32),
        pltpu.VMEM_SHARED((N, HP), jnp.float32),
        pltpu.SemaphoreType.DMA,
        pltpu.SemaphoreType.DMA,
        pltpu.SemaphoreType.DMA,
        pltpu.SemaphoreType.DMA,
        pltpu.SemaphoreType.DMA,
    ],
)
def _edge_scores(q_hbm, k_hbm, dsts_hbm, srcs_hbm, zero4_hbm,
                 ex_hbm, den_hbm,
                 dsti, srci, qbuf, kbuf, exbuf, den_sp,
                 semg0, semg1, sems0, sems1, seme):
    cid = lax.axis_index("c")
    sid = lax.axis_index("s")
    wid = sid * NC + cid
    semg = (semg0, semg1)
    sems = (sems0, sems1)

    @pl.when(sid < NWB)
    def _init():
        pltpu.sync_copy(zero4_hbm, den_sp.at[pl.ds(sid * NPS, NPS)])

    lane = lax.iota(jnp.int32, L)
    pltpu.sync_copy(dsts_hbm.at[pl.ds(wid * RPW, RPW)], dsti)
    pltpu.sync_copy(srcs_hbm.at[pl.ds(wid * RPW, RPW)], srci)
    plsc.subcore_barrier()

    def fire_gathers(i, b):
        for j in range(SUB):
            row = i * SUB + j
            pltpu.async_copy(q_hbm.at[dsti.at[row]],
                             qbuf.at[b, pl.ds(j * SB, SB)], semg[b])
            pltpu.async_copy(k_hbm.at[srci.at[row]],
                             kbuf.at[b, pl.ds(j * SB, SB)], semg[b])

    def compute(i, b):
        @plsc.parallel_loop(0, CB, step=1, unroll=4)
        def edge(e):
            p = [qbuf[b, e, pl.ds(16 * t, L)] * kbuf[b, e, pl.ds(16 * t, L)]
                 for t in range(D // L)]
            sv = jnp.zeros((L,), jnp.float32)
            for h in range(H):
                sv = jnp.where(lane == h,
                               jnp.full((L,), jnp.sum(p[2 * h] + p[2 * h + 1])),
                               sv)
            exbuf[b * CB + e] = jnp.where(lane < H, jnp.exp(sv), 0.0)

    def fire_adds(i, b):
        for j in range(SUB):
            pltpu.async_copy(exbuf.at[pl.ds(b * CB + j * SB, SB)],
                             den_sp.at[dsti.at[i * SUB + j]], sems[b],
                             add=True)

    def drain_gathers(b):
        for j in range(SUB):
            _drain(semg[b], q_hbm.at[pl.ds(0, SB)],
                   qbuf.at[b, pl.ds(j * SB, SB)])
            _drain(semg[b], q_hbm.at[pl.ds(0, SB)],
                   kbuf.at[b, pl.ds(j * SB, SB)])

    def drain_adds(b):
        for j in range(SUB):
            _drain(sems[b], ex_hbm.at[pl.ds(0, SB)],
                   exbuf.at[pl.ds(b * CB + j * SB, SB)])

    fire_gathers(0, 0)

    def pair(jp, carry):
        i0 = jp * 2
        fire_gathers(i0 + 1, 1)

        @pl.when(jp >= 1)
        def _d0():
            drain_adds(0)
            _drain(seme, ex_hbm.at[pl.ds(0, 2 * CB)], exbuf)

        drain_gathers(0)
        compute(i0, 0)
        fire_adds(i0, 0)
        @pl.when(jp <= NPAIR - 2)
        def _g1():
            fire_gathers(i0 + 2, 0)

        @pl.when(jp >= 1)
        def _d1():
            drain_adds(1)

        drain_gathers(1)
        compute(i0 + 1, 1)
        fire_adds(i0 + 1, 1)
        pltpu.async_copy(exbuf, ex_hbm.at[pl.ds(wid * EPW + i0 * CB, 2 * CB)],
                         seme)
        return carry

    lax.fori_loop(0, NPAIR, pair, 0)
    drain_adds(0)
    drain_adds(1)
    _drain(seme, ex_hbm.at[pl.ds(0, 2 * CB)], exbuf)
    plsc.subcore_barrier()

    @pl.when(sid < NWB)
    def _writeback():
        pltpu.sync_copy(den_sp.at[pl.ds(sid * NPS, NPS)],
                        den_hbm.at[pl.ds(cid * N + sid * NPS, NPS)])


@functools.partial(
    pl.kernel,
    out_type=jax.ShapeDtypeStruct((NC * N, D), jnp.float32),
    mesh=_mesh,
    compiler_params=_sc_params,
    scratch_types=[
        pltpu.VMEM((RPW, SB), jnp.int32),
        pltpu.VMEM((RPW, SB), jnp.int32),
        pltpu.VMEM((2, CB_B, D), jnp.float32),
        pltpu.VMEM((2, CB_B, D), jnp.float32),
        pltpu.VMEM((2 * CB_B * HP,), jnp.float32),
        pltpu.VMEM_SHARED((N, D), jnp.float32),
        pltpu.SemaphoreType.DMA,
        pltpu.SemaphoreType.DMA,
        pltpu.SemaphoreType.DMA,
        pltpu.SemaphoreType.DMA,
    ],
)
def _aggregate(v_hbm, dsts_hbm, srcs_hbm, exflat_hbm, zero128_hbm,
               agg_hbm,
               dsti, srci, vbuf, msgbuf, exbuf, agg_sp,
               semg0, semg1, sems0, sems1):
    cid = lax.axis_index("c")
    sid = lax.axis_index("s")
    wid = sid * NC + cid
    semg = (semg0, semg1)
    sems = (sems0, sems1)

    @pl.when(sid < NWB)
    def _init():
        pltpu.sync_copy(zero128_hbm, agg_sp.at[pl.ds(sid * NPS, NPS)])

    pltpu.sync_copy(dsts_hbm.at[pl.ds(wid * RPW, RPW)], dsti)
    pltpu.sync_copy(srcs_hbm.at[pl.ds(wid * RPW, RPW)], srci)
    plsc.subcore_barrier()

    def fire_gathers(i, b):
        for j in range(SUB_B):
            pltpu.async_copy(v_hbm.at[srci.at[i * SUB_B + j]],
                             vbuf.at[b, pl.ds(j * SB, SB)], semg[b])
        pltpu.async_copy(
            exflat_hbm.at[pl.ds((wid * EPW + i * CB_B) * HP, CB_B * HP)],
            exbuf.at[pl.ds(b * CB_B * HP, CB_B * HP)], semg[b])

    def compute(i, b):
        @plsc.parallel_loop(0, CB_B, step=1, unroll=4)
        def edge(e):
            ev = exbuf[pl.ds((b * CB_B + e) * HP, L)]
            for h in range(H):
                a = jnp.full((L,), ev[h])
                for t2 in range(2):
                    c0 = h * DH + t2 * L
                    msgbuf[b, e, pl.ds(c0, L)] = vbuf[b, e, pl.ds(c0, L)] * a

    def fire_adds(i, b):
        for j in range(SUB_B):
            pltpu.async_copy(msgbuf.at[b, pl.ds(j * SB, SB)],
                             agg_sp.at[dsti.at[i * SUB_B + j]], sems[b],
                             add=True)

    def drain_gathers(b):
        for j in range(SUB_B):
            _drain(semg[b], v_hbm.at[pl.ds(0, SB)],
                   vbuf.at[b, pl.ds(j * SB, SB)])
        _drain(semg[b], exflat_hbm.at[pl.ds(0, CB_B * HP)],
               exbuf.at[pl.ds(b * CB_B * HP, CB_B * HP)])

    def drain_adds(b):
        for j in range(SUB_B):
            _drain(sems[b], v_hbm.at[pl.ds(0, SB)],
                   msgbuf.at[b, pl.ds(j * SB, SB)])

    fire_gathers(0, 0)

    def pair(jp, carry):
        i0 = jp * 2
        fire_gathers(i0 + 1, 1)

        @pl.when(jp >= 1)
        def _d0():
            drain_adds(0)

        drain_gathers(0)
        compute(i0, 0)
        fire_adds(i0, 0)

        @pl.when(jp <= NPAIR_B - 2)
        def _g1():
            fire_gathers(i0 + 2, 0)

        @pl.when(jp >= 1)
        def _d1():
            drain_adds(1)

        drain_gathers(1)
        compute(i0 + 1, 1)
        fire_adds(i0 + 1, 1)
        return carry

    lax.fori_loop(0, NPAIR_B, pair, 0)
    drain_adds(0)
    drain_adds(1)
    plsc.subcore_barrier()

    @pl.when(sid < NWB)
    def _writeback():
        pltpu.sync_copy(agg_sp.at[pl.ds(sid * NPS, NPS)],
                        agg_hbm.at[pl.ds(cid * N + sid * NPS, NPS)])


BR = 1000


def _qkv_body(x_ref, wq_ref, wk_ref, wv_ref, q_ref, k_ref, v_ref):
    xb = x_ref[...]
    q_ref[...] = jnp.dot(xb, wq_ref[...],
                         preferred_element_type=jnp.float32) * INV
    k_ref[...] = jnp.dot(xb, wk_ref[...], preferred_element_type=jnp.float32)
    v_ref[...] = jnp.dot(xb, wv_ref[...], preferred_element_type=jnp.float32)


_qkv_call = pl.pallas_call(
    _qkv_body,
    grid=(N // BR,),
    in_specs=[pl.BlockSpec((BR, D), lambda i: (i, 0))]
    + [pl.BlockSpec((D, D), lambda i: (0, 0))] * 3,
    out_specs=[pl.BlockSpec((BR, D), lambda i: (i, 0))] * 3,
    out_shape=[jax.ShapeDtypeStruct((N, D), jnp.float32)] * 3,
)


def _densum_body(d_ref, o_ref):
    o_ref[...] = d_ref[0] + d_ref[1]


_densum_call = pl.pallas_call(
    _densum_body,
    in_specs=[pl.BlockSpec((NC, N * HP // D, D), lambda: (0, 0, 0))],
    out_specs=pl.BlockSpec((N * HP // D, D), lambda: (0, 0)),
    out_shape=jax.ShapeDtypeStruct((N * HP // D, D), jnp.float32),
)


def _out_body(x_ref, a0_ref, a1_ref, d_ref, wo_ref, o_ref):
    row_h = lax.broadcasted_iota(jnp.int32, (HP, D), 0)
    col_h = lax.broadcasted_iota(jnp.int32, (HP, D), 1) // DH
    bmat = jnp.where(row_h == col_h, 1.0, 0.0).astype(jnp.float32)
    r = 1.0 / (d_ref[...] + 1e-9)
    dens = jnp.dot(r, bmat, preferred_element_type=jnp.float32)
    agg = (a0_ref[...] + a1_ref[...]) * dens
    o_ref[...] = x_ref[...] + jnp.dot(agg, wo_ref[...],
                                      preferred_element_type=jnp.float32)


_out_call = pl.pallas_call(
    _out_body,
    grid=(N // BR,),
    in_specs=[
        pl.BlockSpec((BR, D), lambda i: (i, 0)),
        pl.BlockSpec((BR, D), lambda i: (i, 0)),
        pl.BlockSpec((BR, D), lambda i: (i + N // BR, 0)),
        pl.BlockSpec((BR, HP), lambda i: (i, 0)),
        pl.BlockSpec((D, D), lambda i: (0, 0)),
    ],
    out_specs=pl.BlockSpec((BR, D), lambda i: (i, 0)),
    out_shape=jax.ShapeDtypeStruct((N, D), jnp.float32),
)


def kernel(x, Wq, Wk, Wv, Wo, edge_index):
    q, k, v = _qkv_call(x, Wq, Wk, Wv)
    srcs = edge_index[0].reshape(E // SB, SB)
    dsts = edge_index[1].reshape(E // SB, SB)
    zero4 = jnp.zeros((NPS, HP), jnp.float32)
    zero128 = jnp.zeros((NPS, D), jnp.float32)
    ex, den = _edge_scores(q, k, dsts, srcs, zero4)
    densum = _densum_call(den.reshape(NC, N * HP // D, D)).reshape(N, HP)
    agg = _aggregate(v, dsts, srcs, ex.reshape(-1), zero128)
    return _out_call(x, agg, agg, densum, Wo)
